# baseline (speedup 1.0000x reference)
import functools

import jax
import jax.numpy as jnp
from jax import lax
from jax.experimental import pallas as pl
from jax.experimental.pallas import tpu as pltpu
from jax.experimental.pallas import tpu_sc as plsc

_AT = 10000
_NBR = 32
_E = _AT * _NBR
_F = 128
_GF_END = 5.5
_BA = 80
_EB = _BA * _NBR
_NBLK = _AT // _BA
_CHUNK = 80
_NW = 32
_LN2 = 0.6931471805599453

_F32 = jnp.float32
_BF16 = jnp.bfloat16


def _ssp(x):
    return jnp.maximum(x, 0.0) + jnp.log(1.0 + jnp.exp(-jnp.abs(x))) - _LN2


def _gauss(d):
    width = _GF_END / (_F - 1)
    centers = jnp.arange(_F, dtype=jnp.int32).astype(_F32) * width
    z = (d[:, :, None] - centers[None, None, :]) * (1.0 / width)
    return jnp.exp(-0.5 * z * z)


def _bf_to_i32(x):
    n = x.shape[0]
    return lax.bitcast_convert_type(x.reshape(n, _F, 2), jnp.int32)


def _i32_to_bf(x):
    n = x.shape[0]
    return lax.bitcast_convert_type(x, _BF16).reshape(n, 2 * _F)


def _sc_gather(table, idx):
    n_out = idx.shape[0]
    total_chunks = n_out // _CHUNK
    per_w = -(-total_chunks // _NW)
    mesh = plsc.VectorSubcoreMesh(core_axis_name="c", subcore_axis_name="s")

    @functools.partial(
        pl.kernel,
        out_type=jax.ShapeDtypeStruct((n_out, _F), table.dtype),
        mesh=mesh,
        scratch_types=[
            pltpu.VMEM((_CHUNK,), jnp.int32),
            pltpu.VMEM((_CHUNK, _F), table.dtype),
            pltpu.SemaphoreType.DMA,
        ],
    )
    def gk(table_hbm, idx_hbm, out_hbm, idx_v, rows_v, sem):
        wid = lax.axis_index("s") * 2 + lax.axis_index("c")

        def body(i, carry):
            chunk = wid * per_w + i

            @pl.when(chunk < total_chunks)
            def _():
                base = chunk * _CHUNK
                pltpu.sync_copy(idx_hbm.at[pl.ds(base, _CHUNK)], idx_v)
                pltpu.async_copy(table_hbm.at[idx_v], rows_v, sem).wait()
                pltpu.sync_copy(rows_v, out_hbm.at[pl.ds(base, _CHUNK)])

            return carry

        lax.fori_loop(0, per_w, body, None)

    return gk(table, idx)


def _dot(a, b):
    return jnp.dot(a, b, preferred_element_type=_F32)


def _edge_update(e3, gn32, h, m3, ewh, ewe, eb):
    a = _dot(h, ewh) + eb
    lin2 = gn32 + _dot(e3.reshape(_EB, _F), ewe)
    lin3 = lin2.reshape(_BA, _NBR, _F) + a[:, None, :]
    return e3 + _ssp(lin3) * m3


def _msg_pass(e3, g32, h, m3, fw, fb, nw, nb):
    filt = _ssp(_dot(e3.reshape(_EB, _F), fw) + fb)
    msg = g32.reshape(_BA, _NBR, _F) * filt.reshape(_BA, _NBR, _F) * m3
    agg = jnp.sum(msg, axis=1)
    return h + _ssp(_dot(agg, nw) + nb)


def _pack_out(h_new, ewn_next):
    n_new = _dot(h_new, ewn_next)
    return jnp.concatenate([h_new.astype(_BF16), n_new.astype(_BF16)],
                           axis=-1)


def _p0_body(d_ref, g_ref, h_ref, m_ref, fw_ref, fb_ref, nw_ref, nb_ref,
             ewn_ref, h_out_ref, pk_out_ref):
    e3 = _gauss(d_ref[...])
    m3 = m_ref[...][:, :, None]
    g32 = g_ref[...]
    h_new = _msg_pass(e3, g32, h_ref[...], m3, fw_ref[...], fb_ref[...],
                      nw_ref[...], nb_ref[...])
    h_out_ref[...] = h_new
    pk_out_ref[...] = _pack_out(h_new, ewn_ref[...])


def _pmid_body(first, e_ref, g_ref, h_ref, m_ref,
               ewh_ref, ewe_ref, eb_ref,
               fw_ref, fb_ref, nw_ref, nb_ref, ewn_ref,
               e_out_ref, h_out_ref, pk_out_ref):
    if first:
        e3 = _gauss(e_ref[...])
    else:
        e3 = e_ref[...].astype(_F32).reshape(_BA, _NBR, _F)
    m3 = m_ref[...][:, :, None]
    pk = g_ref[...]
    g32 = pk[:, :_F].astype(_F32)
    gn32 = pk[:, _F:].astype(_F32)
    h = h_ref[...]
    e_new = _edge_update(e3, gn32, h, m3, ewh_ref[...], ewe_ref[...],
                         eb_ref[...])
    e_out_ref[...] = e_new.reshape(_EB, _F).astype(_BF16)
    h_new = _msg_pass(e_new, g32, h, m3, fw_ref[...], fb_ref[...],
                      nw_ref[...], nb_ref[...])
    h_out_ref[...] = h_new
    pk_out_ref[...] = _pack_out(h_new, ewn_ref[...])


def _pfin_body(e_ref, g_ref, h_ref, m_ref, u_ref,
               ewh_ref, ewe_ref, eb_ref,
               ow1_ref, ob1_ref, ow2_ref, ob2_ref,
               f_out_ref):
    e3 = e_ref[...].astype(_F32).reshape(_BA, _NBR, _F)
    m3 = m_ref[...][:, :, None]
    gn32 = g_ref[...][:, _F:].astype(_F32)
    e_new = _edge_update(e3, gn32, h_ref[...], m3, ewh_ref[...],
                         ewe_ref[...], eb_ref[...])
    t = _ssp(_dot(e_new.reshape(_EB, _F), ow1_ref[...]) + ob1_ref[...])
    fm = _dot(t, ow2_ref[...]) + ob2_ref[...]
    f_out_ref[...] = jnp.sum(fm.reshape(_BA, _NBR, 1) * u_ref[...], axis=1)


def _spec_w(shape):
    nd = len(shape)
    return pl.BlockSpec(shape, lambda i, _n=nd: (0,) * _n)


_SPEC_D = pl.BlockSpec((_BA, _NBR), lambda i: (i, 0))
_SPEC_E = pl.BlockSpec((_EB, _F), lambda i: (i, 0))
_SPEC_G = pl.BlockSpec((_EB, 2 * _F), lambda i: (i, 0))
_SPEC_H = pl.BlockSpec((_BA, _F), lambda i: (i, 0))
_SPEC_PK = pl.BlockSpec((_BA, 2 * _F), lambda i: (i, 0))
_SPEC_U = pl.BlockSpec((_BA, _NBR, 3), lambda i: (i, 0, 0))
_SPEC_F = pl.BlockSpec((_BA, 3), lambda i: (i, 0))
_PARAMS = pltpu.CompilerParams(dimension_semantics=("arbitrary",))


def _pass0(d2, g0, h0, m2, fw, fb, nw, nb, ewn_next):
    return pl.pallas_call(
        _p0_body,
        grid=(_NBLK,),
        in_specs=[_SPEC_D, _SPEC_E, _SPEC_H, _SPEC_D,
                  _spec_w((_F, _F)), _spec_w((1, _F)),
                  _spec_w((_F, _F)), _spec_w((1, _F)),
                  _spec_w((_F, _F))],
        out_specs=[_SPEC_H, _SPEC_PK],
        out_shape=[jax.ShapeDtypeStruct((_AT, _F), _F32),
                   jax.ShapeDtypeStruct((_AT, 2 * _F), _BF16)],
        compiler_params=_PARAMS,
    )(d2, g0, h0, m2, fw, fb, nw, nb, ewn_next)


def _pass_mid(first, e_in, g, h, m2, ewh, ewe, eb, fw, fb, nw, nb, ewn_next):
    e_spec = _SPEC_D if first else _SPEC_E
    return pl.pallas_call(
        functools.partial(_pmid_body, first),
        grid=(_NBLK,),
        in_specs=[e_spec, _SPEC_G, _SPEC_H, _SPEC_D,
                  _spec_w((_F, _F)), _spec_w((_F, _F)), _spec_w((1, _F)),
                  _spec_w((_F, _F)), _spec_w((1, _F)),
                  _spec_w((_F, _F)), _spec_w((1, _F)),
                  _spec_w((_F, _F))],
        out_specs=[_SPEC_E, _SPEC_H, _SPEC_PK],
        out_shape=[jax.ShapeDtypeStruct((_E, _F), _BF16),
                   jax.ShapeDtypeStruct((_AT, _F), _F32),
                   jax.ShapeDtypeStruct((_AT, 2 * _F), _BF16)],
        compiler_params=_PARAMS,
    )(e_in, g, h, m2, ewh, ewe, eb, fw, fb, nw, nb, ewn_next)


def _pass_fin(e_in, g, h, m2, u3, ewh, ewe, eb, ow1, ob1, ow2, ob2):
    return pl.pallas_call(
        _pfin_body,
        grid=(_NBLK,),
        in_specs=[_SPEC_E, _SPEC_G, _SPEC_H, _SPEC_D, _SPEC_U,
                  _spec_w((_F, _F)), _spec_w((_F, _F)), _spec_w((1, _F)),
                  _spec_w((_F, _F // 2)), _spec_w((1, _F // 2)),
                  _spec_w((_F // 2, 1)), _spec_w((1, 1))],
        out_specs=_SPEC_F,
        out_shape=jax.ShapeDtypeStruct((_AT, 3), _F32),
        compiler_params=_PARAMS,
    )(e_in, g, h, m2, u3, ewh, ewe, eb, ow1, ob1, ow2, ob2)


def kernel(Z, distances, neighbors, neighbor_mask, unit_vecs, params):
    zf = Z.reshape(_AT).astype(jnp.int32)
    nb_flat = neighbors.reshape(_E).astype(jnp.int32)
    d2 = distances.reshape(_AT, _NBR)
    m2 = neighbor_mask.reshape(_AT, _NBR)
    u3 = unit_vecs.reshape(_AT, _NBR, 3)
    ls = params["layers"]

    def w(l):
        p = ls[l]
        ew = p["ew"]
        return (ew[:_F], ew[_F:2 * _F], ew[2 * _F:],
                p["eb"].reshape(1, _F), p["fw"], p["fb"].reshape(1, _F),
                p["nw"], p["nb"].reshape(1, _F))

    def gather_pk(pk_bf):
        return _i32_to_bf(_sc_gather(_bf_to_i32(pk_bf), nb_flat))

    ewh0, ewn0, ewe0, eb0 = w(0)[:4]
    ewh1, ewn1, ewe1, eb1 = w(1)[:4]
    ewh2, ewn2, ewe2, eb2 = w(2)[:4]
    fw1, fb1, nw1, nb1 = w(1)[4:]
    fw2, fb2, nw2, nb2 = w(2)[4:]

    h0 = _sc_gather(params["embed"], zf)
    g0 = _sc_gather(h0, nb_flat)
    h1, pk1 = _pass0(d2, g0, h0, m2, ls[0]["fw"], ls[0]["fb"].reshape(1, _F),
                     ls[0]["nw"], ls[0]["nb"].reshape(1, _F), ewn0)

    g1 = gather_pk(pk1)
    e1, h2, pk2 = _pass_mid(True, d2, g1, h1, m2, ewh0, ewe0, eb0,
                            fw1, fb1, nw1, nb1, ewn1)
    g2 = gather_pk(pk2)
    e2, h3, pk3 = _pass_mid(False, e1, g2, h2, m2, ewh1, ewe1, eb1,
                            fw2, fb2, nw2, nb2, ewn2)
    g3 = gather_pk(pk3)
    forces = _pass_fin(e2, g3, h3, m2, u3, ewh2, ewe2, eb2,
                       params["ow1"], params["ob1"].reshape(1, _F // 2),
                       params["ow2"], params["ob2"].reshape(1, 1))
    return forces.reshape(1, _AT, 3)


# speedup vs baseline: 3.0380x; 3.0380x over previous
import functools

import jax
import jax.numpy as jnp
from jax import lax
from jax.experimental import pallas as pl
from jax.experimental.pallas import tpu as pltpu
from jax.experimental.pallas import tpu_sc as plsc

_AT = 10000
_NBR = 32
_E = _AT * _NBR
_F = 128
_GF_END = 5.5
_BA = 80
_EB = _BA * _NBR
_NBLK = _AT // _BA
_CHUNK = 80
_NW = 32
_LN2 = 0.6931471805599453

_F32 = jnp.float32
_BF16 = jnp.bfloat16


def _ssp(x):
    return jnp.maximum(x, 0.0) + jnp.log(1.0 + jnp.exp(-jnp.abs(x))) - _LN2


def _gauss(d):
    width = _GF_END / (_F - 1)
    centers = jnp.arange(_F, dtype=jnp.int32).astype(_F32) * width
    z = (d[:, :, None] - centers[None, None, :]) * (1.0 / width)
    return jnp.exp(-0.5 * z * z)


def _unpack_hi(pk):
    return lax.bitcast_convert_type(pk & jnp.uint32(0xFFFF0000), _F32)


def _unpack_lo(pk):
    return lax.bitcast_convert_type(pk << 16, _F32)


def _sc_gather(table, idx):
    n_out = idx.shape[0]
    total_chunks = n_out // _CHUNK
    per_w = -(-total_chunks // _NW)
    mesh = plsc.VectorSubcoreMesh(core_axis_name="c", subcore_axis_name="s")

    @functools.partial(
        pl.kernel,
        out_type=jax.ShapeDtypeStruct((n_out, _F), table.dtype),
        mesh=mesh,
        scratch_types=[
            pltpu.VMEM((_CHUNK,), jnp.int32),
            pltpu.VMEM((_CHUNK, _F), table.dtype),
            pltpu.SemaphoreType.DMA,
        ],
    )
    def gk(table_hbm, idx_hbm, out_hbm, idx_v, rows_v, sem):
        wid = lax.axis_index("s") * 2 + lax.axis_index("c")

        def body(i, carry):
            chunk = wid * per_w + i

            @pl.when(chunk < total_chunks)
            def _():
                base = chunk * _CHUNK
                pltpu.sync_copy(idx_hbm.at[pl.ds(base, _CHUNK)], idx_v)
                pltpu.async_copy(table_hbm.at[idx_v], rows_v, sem).wait()
                pltpu.sync_copy(rows_v, out_hbm.at[pl.ds(base, _CHUNK)])

            return carry

        lax.fori_loop(0, per_w, body, None)

    return gk(table, idx)


def _dot(a, b):
    return jnp.dot(a, b, preferred_element_type=_F32)


def _edge_update(e3, gn32, h, m3, ewh, ewe, eb):
    a = _dot(h, ewh) + eb
    lin2 = gn32 + _dot(e3.reshape(_EB, _F), ewe)
    lin3 = lin2.reshape(_BA, _NBR, _F) + a[:, None, :]
    return e3 + _ssp(lin3) * m3


def _msg_pass(e3, g32, h, m3, fw, fb, nw, nb):
    filt = _ssp(_dot(e3.reshape(_EB, _F), fw) + fb)
    msg = g32.reshape(_BA, _NBR, _F) * filt.reshape(_BA, _NBR, _F) * m3
    agg = jnp.sum(msg, axis=1)
    return h + _ssp(_dot(agg, nw) + nb)


def _pack_out(h_new, ewn_next):
    n_new = _dot(h_new, ewn_next)
    hb = lax.bitcast_convert_type(h_new, jnp.uint32)
    nb_ = lax.bitcast_convert_type(n_new, jnp.uint32)
    hr = (hb + jnp.uint32(0x8000)) & jnp.uint32(0xFFFF0000)
    nr = (nb_ + jnp.uint32(0x8000)) >> 16
    return hr | nr


def _p0_body(d_ref, g_ref, h_ref, m_ref, fw_ref, fb_ref, nw_ref, nb_ref,
             ewn_ref, h_out_ref, pk_out_ref):
    e3 = _gauss(d_ref[...])
    m3 = m_ref[...][:, :, None]
    g32 = g_ref[...]
    h_new = _msg_pass(e3, g32, h_ref[...], m3, fw_ref[...], fb_ref[...],
                      nw_ref[...], nb_ref[...])
    h_out_ref[...] = h_new
    pk_out_ref[...] = _pack_out(h_new, ewn_ref[...])


def _pmid_body(first, e_ref, g_ref, h_ref, m_ref,
               ewh_ref, ewe_ref, eb_ref,
               fw_ref, fb_ref, nw_ref, nb_ref, ewn_ref,
               e_out_ref, h_out_ref, pk_out_ref):
    if first:
        e3 = _gauss(e_ref[...])
    else:
        e3 = e_ref[...].astype(_F32).reshape(_BA, _NBR, _F)
    m3 = m_ref[...][:, :, None]
    pk = g_ref[...]
    g32 = _unpack_hi(pk)
    gn32 = _unpack_lo(pk)
    h = h_ref[...]
    e_new = _edge_update(e3, gn32, h, m3, ewh_ref[...], ewe_ref[...],
                         eb_ref[...])
    e_out_ref[...] = e_new.reshape(_EB, _F).astype(_BF16)
    h_new = _msg_pass(e_new, g32, h, m3, fw_ref[...], fb_ref[...],
                      nw_ref[...], nb_ref[...])
    h_out_ref[...] = h_new
    pk_out_ref[...] = _pack_out(h_new, ewn_ref[...])


def _pfin_body(e_ref, g_ref, h_ref, m_ref, u_ref,
               ewh_ref, ewe_ref, eb_ref,
               ow1_ref, ob1_ref, ow2_ref, ob2_ref,
               f_out_ref):
    e3 = e_ref[...].astype(_F32).reshape(_BA, _NBR, _F)
    m3 = m_ref[...][:, :, None]
    gn32 = _unpack_lo(g_ref[...])
    e_new = _edge_update(e3, gn32, h_ref[...], m3, ewh_ref[...],
                         ewe_ref[...], eb_ref[...])
    t = _ssp(_dot(e_new.reshape(_EB, _F), ow1_ref[...]) + ob1_ref[...])
    fm = _dot(t, ow2_ref[...]) + ob2_ref[...]
    f_out_ref[...] = jnp.sum(fm.reshape(_BA, _NBR, 1) * u_ref[...], axis=1)


def _spec_w(shape):
    nd = len(shape)
    return pl.BlockSpec(shape, lambda i, _n=nd: (0,) * _n)


_SPEC_D = pl.BlockSpec((_BA, _NBR), lambda i: (i, 0))
_SPEC_E = pl.BlockSpec((_EB, _F), lambda i: (i, 0))
_SPEC_G = pl.BlockSpec((_EB, _F), lambda i: (i, 0))
_SPEC_H = pl.BlockSpec((_BA, _F), lambda i: (i, 0))
_SPEC_PK = pl.BlockSpec((_BA, _F), lambda i: (i, 0))
_SPEC_U = pl.BlockSpec((_BA, _NBR, 3), lambda i: (i, 0, 0))
_SPEC_F = pl.BlockSpec((_BA, 3), lambda i: (i, 0))
_PARAMS = pltpu.CompilerParams(dimension_semantics=("arbitrary",))


def _pass0(d2, g0, h0, m2, fw, fb, nw, nb, ewn_next):
    return pl.pallas_call(
        _p0_body,
        grid=(_NBLK,),
        in_specs=[_SPEC_D, _SPEC_E, _SPEC_H, _SPEC_D,
                  _spec_w((_F, _F)), _spec_w((1, _F)),
                  _spec_w((_F, _F)), _spec_w((1, _F)),
                  _spec_w((_F, _F))],
        out_specs=[_SPEC_H, _SPEC_PK],
        out_shape=[jax.ShapeDtypeStruct((_AT, _F), _F32),
                   jax.ShapeDtypeStruct((_AT, _F), jnp.uint32)],
        compiler_params=_PARAMS,
    )(d2, g0, h0, m2, fw, fb, nw, nb, ewn_next)


def _pass_mid(first, e_in, g, h, m2, ewh, ewe, eb, fw, fb, nw, nb, ewn_next):
    e_spec = _SPEC_D if first else _SPEC_E
    return pl.pallas_call(
        functools.partial(_pmid_body, first),
        grid=(_NBLK,),
        in_specs=[e_spec, _SPEC_G, _SPEC_H, _SPEC_D,
                  _spec_w((_F, _F)), _spec_w((_F, _F)), _spec_w((1, _F)),
                  _spec_w((_F, _F)), _spec_w((1, _F)),
                  _spec_w((_F, _F)), _spec_w((1, _F)),
                  _spec_w((_F, _F))],
        out_specs=[_SPEC_E, _SPEC_H, _SPEC_PK],
        out_shape=[jax.ShapeDtypeStruct((_E, _F), _BF16),
                   jax.ShapeDtypeStruct((_AT, _F), _F32),
                   jax.ShapeDtypeStruct((_AT, _F), jnp.uint32)],
        compiler_params=_PARAMS,
    )(e_in, g, h, m2, ewh, ewe, eb, fw, fb, nw, nb, ewn_next)


def _pass_fin(e_in, g, h, m2, u3, ewh, ewe, eb, ow1, ob1, ow2, ob2):
    return pl.pallas_call(
        _pfin_body,
        grid=(_NBLK,),
        in_specs=[_SPEC_E, _SPEC_G, _SPEC_H, _SPEC_D, _SPEC_U,
                  _spec_w((_F, _F)), _spec_w((_F, _F)), _spec_w((1, _F)),
                  _spec_w((_F, _F // 2)), _spec_w((1, _F // 2)),
                  _spec_w((_F // 2, 1)), _spec_w((1, 1))],
        out_specs=_SPEC_F,
        out_shape=jax.ShapeDtypeStruct((_AT, 3), _F32),
        compiler_params=_PARAMS,
    )(e_in, g, h, m2, u3, ewh, ewe, eb, ow1, ob1, ow2, ob2)


def kernel(Z, distances, neighbors, neighbor_mask, unit_vecs, params):
    zf = Z.reshape(_AT).astype(jnp.int32)
    nb_flat = neighbors.reshape(_E).astype(jnp.int32)
    d2 = distances.reshape(_AT, _NBR)
    m2 = neighbor_mask.reshape(_AT, _NBR)
    u3 = unit_vecs.reshape(_AT, _NBR, 3)
    ls = params["layers"]

    def w(l):
        p = ls[l]
        ew = p["ew"]
        return (ew[:_F], ew[_F:2 * _F], ew[2 * _F:],
                p["eb"].reshape(1, _F), p["fw"], p["fb"].reshape(1, _F),
                p["nw"], p["nb"].reshape(1, _F))

    def gather_pk(pk_u32):
        return _sc_gather(pk_u32, nb_flat)

    ewh0, ewn0, ewe0, eb0 = w(0)[:4]
    ewh1, ewn1, ewe1, eb1 = w(1)[:4]
    ewh2, ewn2, ewe2, eb2 = w(2)[:4]
    fw1, fb1, nw1, nb1 = w(1)[4:]
    fw2, fb2, nw2, nb2 = w(2)[4:]

    h0 = _sc_gather(params["embed"], zf)
    g0 = _sc_gather(h0, nb_flat)
    h1, pk1 = _pass0(d2, g0, h0, m2, ls[0]["fw"], ls[0]["fb"].reshape(1, _F),
                     ls[0]["nw"], ls[0]["nb"].reshape(1, _F), ewn0)

    g1 = gather_pk(pk1)
    e1, h2, pk2 = _pass_mid(True, d2, g1, h1, m2, ewh0, ewe0, eb0,
                            fw1, fb1, nw1, nb1, ewn1)
    g2 = gather_pk(pk2)
    e2, h3, pk3 = _pass_mid(False, e1, g2, h2, m2, ewh1, ewe1, eb1,
                            fw2, fb2, nw2, nb2, ewn2)
    g3 = gather_pk(pk3)
    forces = _pass_fin(e2, g3, h3, m2, u3, ewh2, ewe2, eb2,
                       params["ow1"], params["ob1"].reshape(1, _F // 2),
                       params["ow2"], params["ob2"].reshape(1, 1))
    return forces.reshape(1, _AT, 3)


# speedup vs baseline: 3.8922x; 1.2812x over previous
import functools

import jax
import jax.numpy as jnp
from jax import lax
from jax.experimental import pallas as pl
from jax.experimental.pallas import tpu as pltpu
from jax.experimental.pallas import tpu_sc as plsc

_AT = 10000
_NBR = 32
_E = _AT * _NBR
_F = 128
_GF_END = 5.5
_BA = 80
_EB = _BA * _NBR
_NBLK = _AT // _BA
_CHUNK = 80
_NW = 32
_LN2 = 0.6931471805599453

_F32 = jnp.float32
_BF16 = jnp.bfloat16


def _ssp(x):
    return jnp.maximum(x, 0.0) + jnp.log(1.0 + jnp.exp(-jnp.abs(x))) - _LN2


def _gauss(d):
    width = _GF_END / (_F - 1)
    centers = jnp.arange(_F, dtype=jnp.int32).astype(_F32) * width
    z = (d[:, :, None] - centers[None, None, :]) * (1.0 / width)
    return jnp.exp(-0.5 * z * z)


def _unpack_hi(pk):
    return lax.bitcast_convert_type(pk & jnp.uint32(0xFFFF0000), _F32)


def _unpack_lo(pk):
    return lax.bitcast_convert_type(pk << 16, _F32)


def _sc_gather(table, idx):
    n_out = idx.shape[0]
    total_chunks = n_out // _CHUNK
    per_w = -(-total_chunks // _NW)
    mesh = plsc.VectorSubcoreMesh(core_axis_name="c", subcore_axis_name="s")

    @functools.partial(
        pl.kernel,
        out_type=jax.ShapeDtypeStruct((n_out, _F), table.dtype),
        mesh=mesh,
        scratch_types=[
            pltpu.VMEM((_CHUNK,), jnp.int32),
            pltpu.VMEM((_CHUNK,), jnp.int32),
            pltpu.VMEM((_CHUNK, _F), table.dtype),
            pltpu.VMEM((_CHUNK, _F), table.dtype),
            pltpu.SemaphoreType.DMA,
            pltpu.SemaphoreType.DMA,
        ],
    )
    def gk(table_hbm, idx_hbm, out_hbm, idxa, idxb, rows0, rows1,
           sem0, sem1):
        wid = lax.axis_index("s") * 2 + lax.axis_index("c")
        nvalid = jnp.clip(total_chunks - wid * per_w, 0, per_w)

        def fetch_idx(i, idxv):
            base = (wid * per_w + i) * _CHUNK
            pltpu.sync_copy(idx_hbm.at[pl.ds(base, _CHUNK)], idxv)

        def start(idxv, rows, sem):
            pltpu.async_copy(table_hbm.at[idxv], rows, sem)

        def finish(i, idxv, rows, sem):
            pltpu.make_async_copy(table_hbm.at[idxv], rows, sem).wait()
            base = (wid * per_w + i) * _CHUNK
            pltpu.sync_copy(rows, out_hbm.at[pl.ds(base, _CHUNK)])

        @pl.when(nvalid > 0)
        def _():
            fetch_idx(0, idxa)
            start(idxa, rows0, sem0)

        def body(i, carry):
            @pl.when(i < nvalid)
            def _():
                @pl.when(i % 2 == 0)
                def _():
                    @pl.when(i + 1 < nvalid)
                    def _():
                        fetch_idx(i + 1, idxb)
                        start(idxb, rows1, sem1)
                    finish(i, idxa, rows0, sem0)

                @pl.when(i % 2 == 1)
                def _():
                    @pl.when(i + 1 < nvalid)
                    def _():
                        fetch_idx(i + 1, idxa)
                        start(idxa, rows0, sem0)
                    finish(i, idxb, rows1, sem1)

            return carry

        lax.fori_loop(0, per_w, body, None)

    return gk(table, idx)


def _dot(a, b):
    return jnp.dot(a, b, preferred_element_type=_F32)


def _edge_update(e3, gn32, h, m3, ewh, ewe, eb):
    a = _dot(h, ewh) + eb
    lin2 = gn32 + _dot(e3.reshape(_EB, _F), ewe)
    lin3 = lin2.reshape(_BA, _NBR, _F) + a[:, None, :]
    return e3 + _ssp(lin3) * m3


def _msg_pass(e3, g32, h, m3, fw, fb, nw, nb):
    filt = _ssp(_dot(e3.reshape(_EB, _F), fw) + fb)
    msg = g32.reshape(_BA, _NBR, _F) * filt.reshape(_BA, _NBR, _F) * m3
    agg = jnp.sum(msg, axis=1)
    return h + _ssp(_dot(agg, nw) + nb)


def _pack_out(h_new, ewn_next):
    n_new = _dot(h_new, ewn_next)
    hb = lax.bitcast_convert_type(h_new, jnp.uint32)
    nb_ = lax.bitcast_convert_type(n_new, jnp.uint32)
    hr = (hb + jnp.uint32(0x8000)) & jnp.uint32(0xFFFF0000)
    nr = (nb_ + jnp.uint32(0x8000)) >> 16
    return hr | nr


def _p0_body(d_ref, g_ref, h_ref, m_ref, fw_ref, fb_ref, nw_ref, nb_ref,
             ewn_ref, h_out_ref, pk_out_ref):
    e3 = _gauss(d_ref[...])
    m3 = m_ref[...][:, :, None]
    g32 = g_ref[...]
    h_new = _msg_pass(e3, g32, h_ref[...], m3, fw_ref[...], fb_ref[...],
                      nw_ref[...], nb_ref[...])
    h_out_ref[...] = h_new
    pk_out_ref[...] = _pack_out(h_new, ewn_ref[...])


def _pmid_body(first, e_ref, g_ref, h_ref, m_ref,
               ewh_ref, ewe_ref, eb_ref,
               fw_ref, fb_ref, nw_ref, nb_ref, ewn_ref,
               e_out_ref, h_out_ref, pk_out_ref):
    if first:
        e3 = _gauss(e_ref[...])
    else:
        e3 = e_ref[...].astype(_F32).reshape(_BA, _NBR, _F)
    m3 = m_ref[...][:, :, None]
    pk = g_ref[...]
    g32 = _unpack_hi(pk)
    gn32 = _unpack_lo(pk)
    h = h_ref[...]
    e_new = _edge_update(e3, gn32, h, m3, ewh_ref[...], ewe_ref[...],
                         eb_ref[...])
    e_out_ref[...] = e_new.reshape(_EB, _F).astype(_BF16)
    h_new = _msg_pass(e_new, g32, h, m3, fw_ref[...], fb_ref[...],
                      nw_ref[...], nb_ref[...])
    h_out_ref[...] = h_new
    pk_out_ref[...] = _pack_out(h_new, ewn_ref[...])


def _pfin_body(e_ref, g_ref, h_ref, m_ref, u_ref,
               ewh_ref, ewe_ref, eb_ref,
               ow1_ref, ob1_ref, ow2_ref, ob2_ref,
               f_out_ref):
    e3 = e_ref[...].astype(_F32).reshape(_BA, _NBR, _F)
    m3 = m_ref[...][:, :, None]
    gn32 = _unpack_lo(g_ref[...])
    e_new = _edge_update(e3, gn32, h_ref[...], m3, ewh_ref[...],
                         ewe_ref[...], eb_ref[...])
    t = _ssp(_dot(e_new.reshape(_EB, _F), ow1_ref[...]) + ob1_ref[...])
    fm = _dot(t, ow2_ref[...]) + ob2_ref[...]
    f_out_ref[...] = jnp.sum(fm.reshape(_BA, _NBR, 1) * u_ref[...], axis=1)


def _spec_w(shape):
    nd = len(shape)
    return pl.BlockSpec(shape, lambda i, _n=nd: (0,) * _n)


_SPEC_D = pl.BlockSpec((_BA, _NBR), lambda i: (i, 0))
_SPEC_E = pl.BlockSpec((_EB, _F), lambda i: (i, 0))
_SPEC_G = pl.BlockSpec((_EB, _F), lambda i: (i, 0))
_SPEC_H = pl.BlockSpec((_BA, _F), lambda i: (i, 0))
_SPEC_PK = pl.BlockSpec((_BA, _F), lambda i: (i, 0))
_SPEC_U = pl.BlockSpec((_BA, _NBR, 3), lambda i: (i, 0, 0))
_SPEC_F = pl.BlockSpec((_BA, 3), lambda i: (i, 0))
_PARAMS = pltpu.CompilerParams(dimension_semantics=("arbitrary",))


def _pass0(d2, g0, h0, m2, fw, fb, nw, nb, ewn_next):
    return pl.pallas_call(
        _p0_body,
        grid=(_NBLK,),
        in_specs=[_SPEC_D, _SPEC_E, _SPEC_H, _SPEC_D,
                  _spec_w((_F, _F)), _spec_w((1, _F)),
                  _spec_w((_F, _F)), _spec_w((1, _F)),
                  _spec_w((_F, _F))],
        out_specs=[_SPEC_H, _SPEC_PK],
        out_shape=[jax.ShapeDtypeStruct((_AT, _F), _F32),
                   jax.ShapeDtypeStruct((_AT, _F), jnp.uint32)],
        compiler_params=_PARAMS,
    )(d2, g0, h0, m2, fw, fb, nw, nb, ewn_next)


def _pass_mid(first, e_in, g, h, m2, ewh, ewe, eb, fw, fb, nw, nb, ewn_next):
    e_spec = _SPEC_D if first else _SPEC_E
    return pl.pallas_call(
        functools.partial(_pmid_body, first),
        grid=(_NBLK,),
        in_specs=[e_spec, _SPEC_G, _SPEC_H, _SPEC_D,
                  _spec_w((_F, _F)), _spec_w((_F, _F)), _spec_w((1, _F)),
                  _spec_w((_F, _F)), _spec_w((1, _F)),
                  _spec_w((_F, _F)), _spec_w((1, _F)),
                  _spec_w((_F, _F))],
        out_specs=[_SPEC_E, _SPEC_H, _SPEC_PK],
        out_shape=[jax.ShapeDtypeStruct((_E, _F), _BF16),
                   jax.ShapeDtypeStruct((_AT, _F), _F32),
                   jax.ShapeDtypeStruct((_AT, _F), jnp.uint32)],
        compiler_params=_PARAMS,
    )(e_in, g, h, m2, ewh, ewe, eb, fw, fb, nw, nb, ewn_next)


def _pass_fin(e_in, g, h, m2, u3, ewh, ewe, eb, ow1, ob1, ow2, ob2):
    return pl.pallas_call(
        _pfin_body,
        grid=(_NBLK,),
        in_specs=[_SPEC_E, _SPEC_G, _SPEC_H, _SPEC_D, _SPEC_U,
                  _spec_w((_F, _F)), _spec_w((_F, _F)), _spec_w((1, _F)),
                  _spec_w((_F, _F // 2)), _spec_w((1, _F // 2)),
                  _spec_w((_F // 2, 1)), _spec_w((1, 1))],
        out_specs=_SPEC_F,
        out_shape=jax.ShapeDtypeStruct((_AT, 3), _F32),
        compiler_params=_PARAMS,
    )(e_in, g, h, m2, u3, ewh, ewe, eb, ow1, ob1, ow2, ob2)


def kernel(Z, distances, neighbors, neighbor_mask, unit_vecs, params):
    zf = Z.reshape(_AT).astype(jnp.int32)
    nb_flat = neighbors.reshape(_E).astype(jnp.int32)
    d2 = distances.reshape(_AT, _NBR)
    m2 = neighbor_mask.reshape(_AT, _NBR)
    u3 = unit_vecs.reshape(_AT, _NBR, 3)
    ls = params["layers"]

    def w(l):
        p = ls[l]
        ew = p["ew"]
        return (ew[:_F], ew[_F:2 * _F], ew[2 * _F:],
                p["eb"].reshape(1, _F), p["fw"], p["fb"].reshape(1, _F),
                p["nw"], p["nb"].reshape(1, _F))

    def gather_pk(pk_u32):
        return _sc_gather(pk_u32, nb_flat)

    ewh0, ewn0, ewe0, eb0 = w(0)[:4]
    ewh1, ewn1, ewe1, eb1 = w(1)[:4]
    ewh2, ewn2, ewe2, eb2 = w(2)[:4]
    fw1, fb1, nw1, nb1 = w(1)[4:]
    fw2, fb2, nw2, nb2 = w(2)[4:]

    h0 = _sc_gather(params["embed"], zf)
    g0 = _sc_gather(h0, nb_flat)
    h1, pk1 = _pass0(d2, g0, h0, m2, ls[0]["fw"], ls[0]["fb"].reshape(1, _F),
                     ls[0]["nw"], ls[0]["nb"].reshape(1, _F), ewn0)

    g1 = gather_pk(pk1)
    e1, h2, pk2 = _pass_mid(True, d2, g1, h1, m2, ewh0, ewe0, eb0,
                            fw1, fb1, nw1, nb1, ewn1)
    g2 = gather_pk(pk2)
    e2, h3, pk3 = _pass_mid(False, e1, g2, h2, m2, ewh1, ewe1, eb1,
                            fw2, fb2, nw2, nb2, ewn2)
    g3 = gather_pk(pk3)
    forces = _pass_fin(e2, g3, h3, m2, u3, ewh2, ewe2, eb2,
                       params["ow1"], params["ob1"].reshape(1, _F // 2),
                       params["ow2"], params["ob2"].reshape(1, 1))
    return forces.reshape(1, _AT, 3)
